# baseline (speedup 1.0000x reference)
import functools

import jax
import jax.numpy as jnp
from jax import lax
from jax.experimental import pallas as pl
from jax.experimental.pallas import tpu as pltpu
from jax.experimental.pallas import tpu_sc as plsc

_NC = 2
_NS = 16
_NW = _NC * _NS
_L = 16

_KS = (0, 1, 0 ^ 1 ^ 0x1BD11BDA)
_ROTS = ((13, 15, 26, 6), (17, 29, 16, 24))


def _rotl(x, d):
    return (x << jnp.uint32(d)) | (x >> jnp.uint32(32 - d))


def _threefry_bits(x1):
    x0 = jnp.zeros((_L,), jnp.uint32)
    for r in range(5):
        for d in _ROTS[r % 2]:
            x0 = x0 + x1
            x1 = _rotl(x1, d)
            x1 = x0 ^ x1
        x0 = x0 + jnp.uint32(_KS[(r + 1) % 3] & 0xFFFFFFFF)
        x1 = x1 + jnp.uint32((_KS[(r + 2) % 3] + r + 1) & 0xFFFFFFFF)
    return x0 ^ x1


def _make_sc_call(batch):
    rows_per_tile = batch // _NW
    steps = rows_per_tile // _L
    mesh = plsc.VectorSubcoreMesh(core_axis_name="c", subcore_axis_name="s")

    @functools.partial(
        pl.kernel,
        mesh=mesh,
        out_type=jax.ShapeDtypeStruct((batch, 2), jnp.float32),
        compiler_params=pltpu.CompilerParams(needs_layout_passes=False),
        scratch_types=[
            pltpu.VMEM((_L,), jnp.float32),
            pltpu.VMEM((rows_per_tile, 2), jnp.float32),
        ],
    )
    def sc_call(w_hbm, out_hbm, w_v, out_v):
        wid = lax.axis_index("s") * _NC + lax.axis_index("c")
        base = wid * rows_per_tile

        pltpu.sync_copy(w_hbm, w_v)
        lane = lax.iota(jnp.int32, _L)
        zero_idx = lane * 0
        zero_f = jnp.zeros((_L,), jnp.float32)
        wv = w_v[...]
        w00 = zero_f + wv[0]
        w01 = zero_f + wv[1]
        w10 = zero_f + wv[2]
        w11 = zero_f + wv[3]

        lane_u = lane.astype(jnp.uint32)
        base_u = (base + 1).astype(jnp.uint32)
        col0 = zero_idx
        col1 = zero_idx + 1

        @plsc.parallel_loop(0, steps, 1, unroll=4)
        def step(s):
            bits = _threefry_bits(lane_u + (base_u + s.astype(jnp.uint32) * _L))
            pick1 = (bits >> jnp.uint32(31)) == jnp.uint32(0)
            a = jnp.where(pick1, w01, w00)
            b = jnp.where(pick1, w11, w10)
            rows = lane + s * _L
            plsc.store_scatter(out_v, [rows, col0], a)
            plsc.store_scatter(out_v, [rows, col1], b)

        pltpu.sync_copy(out_v, out_hbm.at[pl.ds(base, rows_per_tile)])

    return sc_call


def kernel(input_ids, attention_mask, W):
    batch = input_ids.shape[0]
    w_flat = jnp.concatenate([W.reshape(4), jnp.zeros((12,), jnp.float32)])
    return _make_sc_call(batch)(w_flat)


# speedup vs baseline: 1.0019x; 1.0019x over previous
import functools

import jax
import jax.numpy as jnp
from jax import lax
from jax.experimental import pallas as pl
from jax.experimental.pallas import tpu as pltpu
from jax.experimental.pallas import tpu_sc as plsc

_NC = 2
_NS = 16
_NW = _NC * _NS
_L = 16

_KS = (0, 1, 0 ^ 1 ^ 0x1BD11BDA)
_ROTS = ((13, 15, 26, 6), (17, 29, 16, 24))


def _rotl(x, d):
    return (x << jnp.uint32(d)) | (x >> jnp.uint32(32 - d))


def _threefry_bits(x1):
    x0 = jnp.zeros((_L,), jnp.uint32)
    for r in range(5):
        for d in _ROTS[r % 2]:
            x0 = x0 + x1
            x1 = _rotl(x1, d)
            x1 = x0 ^ x1
        x0 = x0 + jnp.uint32(_KS[(r + 1) % 3] & 0xFFFFFFFF)
        x1 = x1 + jnp.uint32((_KS[(r + 2) % 3] + r + 1) & 0xFFFFFFFF)
    return x0 ^ x1


def _make_sc_call(batch):
    rows_per_tile = batch // _NW
    steps = rows_per_tile // _L
    mesh = plsc.VectorSubcoreMesh(core_axis_name="c", subcore_axis_name="s")

    @functools.partial(
        pl.kernel,
        mesh=mesh,
        out_type=jax.ShapeDtypeStruct((batch, 2), jnp.float32),
        compiler_params=pltpu.CompilerParams(
            needs_layout_passes=False,
            disable_bounds_checks=True,
            disable_semaphore_checks=True,
            skip_device_barrier=True,
        ),
        scratch_types=[
            pltpu.VMEM((_L,), jnp.float32),
            pltpu.VMEM((rows_per_tile, 2), jnp.float32),
        ],
    )
    def sc_call(w_hbm, out_hbm, w_v, out_v):
        wid = lax.axis_index("s") * _NC + lax.axis_index("c")
        base = wid * rows_per_tile

        pltpu.sync_copy(w_hbm, w_v)
        lane = lax.iota(jnp.int32, _L)
        zero_idx = lane * 0
        zero_f = jnp.zeros((_L,), jnp.float32)
        wv = w_v[...]
        w00 = zero_f + wv[0]
        w01 = zero_f + wv[1]
        w10 = zero_f + wv[2]
        w11 = zero_f + wv[3]

        lane_u = lane.astype(jnp.uint32)
        base_u = (base + 1).astype(jnp.uint32)
        col0 = zero_idx
        col1 = zero_idx + 1

        @plsc.parallel_loop(0, steps, 1, unroll=4)
        def step(s):
            bits = _threefry_bits(lane_u + (base_u + s.astype(jnp.uint32) * _L))
            pick1 = (bits >> jnp.uint32(31)) == jnp.uint32(0)
            a = jnp.where(pick1, w01, w00)
            b = jnp.where(pick1, w11, w10)
            rows = lane + s * _L
            plsc.store_scatter(out_v, [rows, col0], a)
            plsc.store_scatter(out_v, [rows, col1], b)

        pltpu.sync_copy(out_v, out_hbm.at[pl.ds(base, rows_per_tile)])

    return sc_call


def kernel(input_ids, attention_mask, W):
    batch = input_ids.shape[0]
    w_flat = jnp.concatenate([W.reshape(4), jnp.zeros((12,), jnp.float32)])
    return _make_sc_call(batch)(w_flat)


# speedup vs baseline: 1.0217x; 1.0198x over previous
import functools

import jax
import jax.numpy as jnp
from jax import lax
from jax.experimental import pallas as pl
from jax.experimental.pallas import tpu as pltpu
from jax.experimental.pallas import tpu_sc as plsc

_NC = 2
_NS = 16
_NW = _NC * _NS
_L = 16

_KS = (0, 1, 0 ^ 1 ^ 0x1BD11BDA)
_ROTS = ((13, 15, 26, 6), (17, 29, 16, 24))


def _rotl(x, d):
    return (x << jnp.uint32(d)) | (x >> jnp.uint32(32 - d))


def _threefry_bits(x1):
    x0 = jnp.zeros((_L,), jnp.uint32)
    for r in range(5):
        for d in _ROTS[r % 2]:
            x0 = x0 + x1
            x1 = _rotl(x1, d)
            x1 = x0 ^ x1
        x0 = x0 + jnp.uint32(_KS[(r + 1) % 3] & 0xFFFFFFFF)
        x1 = x1 + jnp.uint32((_KS[(r + 2) % 3] + r + 1) & 0xFFFFFFFF)
    return x0 ^ x1


def _make_sc_call(batch):
    rows_per_tile = batch // _NW
    steps = rows_per_tile // _L
    mesh = plsc.VectorSubcoreMesh(core_axis_name="c", subcore_axis_name="s")

    @functools.partial(
        pl.kernel,
        mesh=mesh,
        out_type=jax.ShapeDtypeStruct((batch, 2), jnp.float32),
        compiler_params=pltpu.CompilerParams(needs_layout_passes=False),
        scratch_types=[
            pltpu.VMEM((_L,), jnp.float32),
            pltpu.VMEM((rows_per_tile, 2), jnp.float32),
        ],
    )
    def sc_call(w_hbm, out_hbm, w_v, out_v):
        wid = lax.axis_index("s") * _NC + lax.axis_index("c")
        base = wid * rows_per_tile

        pltpu.sync_copy(w_hbm, w_v)
        lane = lax.iota(jnp.int32, _L)
        zero_idx = lane * 0
        zero_f = jnp.zeros((_L,), jnp.float32)
        wv = w_v[...]
        w00 = zero_f + wv[0]
        w01 = zero_f + wv[1]
        w10 = zero_f + wv[2]
        w11 = zero_f + wv[3]

        lane_u = lane.astype(jnp.uint32)
        base_u = (base + 1).astype(jnp.uint32)
        col0 = zero_idx
        col1 = zero_idx + 1

        def step(s, carry):
            bits = _threefry_bits(lane_u + (base_u + s.astype(jnp.uint32) * _L))
            pick1 = (bits >> jnp.uint32(31)) == jnp.uint32(0)
            a = jnp.where(pick1, w01, w00)
            b = jnp.where(pick1, w11, w10)
            rows = lane + s * _L
            plsc.store_scatter(out_v, [rows, col0], a)
            plsc.store_scatter(out_v, [rows, col1], b)
            return carry

        lax.fori_loop(0, steps, step, 0)

        pltpu.sync_copy(out_v, out_hbm.at[pl.ds(base, rows_per_tile)])

    return sc_call


def kernel(input_ids, attention_mask, W):
    batch = input_ids.shape[0]
    w_flat = jnp.concatenate([W.reshape(4), jnp.zeros((12,), jnp.float32)])
    return _make_sc_call(batch)(w_flat)


# speedup vs baseline: 1.4127x; 1.3827x over previous
import functools

import jax
import jax.numpy as jnp
from jax import lax
from jax.experimental import pallas as pl
from jax.experimental.pallas import tpu as pltpu
from jax.experimental.pallas import tpu_sc as plsc

_NC = 2
_NS = 16
_NW = _NC * _NS
_L = 16

_KS = (0, 1, 0 ^ 1 ^ 0x1BD11BDA)
_ROTS = ((13, 15, 26, 6), (17, 29, 16, 24))


def _rotl(x, d):
    return (x << jnp.uint32(d)) | (x >> jnp.uint32(32 - d))


def _threefry_bits(x1):
    x0 = jnp.zeros((_L,), jnp.uint32)
    for r in range(5):
        for d in _ROTS[r % 2]:
            x0 = x0 + x1
            x1 = _rotl(x1, d)
            x1 = x0 ^ x1
        x0 = x0 + jnp.uint32(_KS[(r + 1) % 3] & 0xFFFFFFFF)
        x1 = x1 + jnp.uint32((_KS[(r + 2) % 3] + r + 1) & 0xFFFFFFFF)
    return x0 ^ x1


def _make_sc_call(batch):
    rows_per_tile = batch // _NW
    steps = rows_per_tile // _L
    mesh = plsc.VectorSubcoreMesh(core_axis_name="c", subcore_axis_name="s")

    @functools.partial(
        pl.kernel,
        mesh=mesh,
        out_type=jax.ShapeDtypeStruct((2, batch), jnp.float32),
        compiler_params=pltpu.CompilerParams(needs_layout_passes=False),
        scratch_types=[
            pltpu.VMEM((_L,), jnp.float32),
            pltpu.VMEM((rows_per_tile,), jnp.float32),
            pltpu.VMEM((rows_per_tile,), jnp.float32),
        ],
    )
    def sc_call(w_hbm, out_hbm, w_v, out_a, out_b):
        wid = lax.axis_index("s") * _NC + lax.axis_index("c")
        base = wid * rows_per_tile

        pltpu.sync_copy(w_hbm, w_v)
        lane = lax.iota(jnp.int32, _L)
        zero_f = jnp.zeros((_L,), jnp.float32)
        wv = w_v[...]
        w00 = zero_f + wv[0]
        w01 = zero_f + wv[1]
        w10 = zero_f + wv[2]
        w11 = zero_f + wv[3]

        lane_u = lane.astype(jnp.uint32)
        base_u = (base + 1).astype(jnp.uint32)

        def step(s, carry):
            bits = _threefry_bits(lane_u + (base_u + s.astype(jnp.uint32) * _L))
            pick1 = (bits >> jnp.uint32(31)) == jnp.uint32(0)
            off = pl.ds(s * _L, _L)
            out_a[off] = jnp.where(pick1, w01, w00)
            out_b[off] = jnp.where(pick1, w11, w10)
            return carry

        lax.fori_loop(0, steps, step, 0)

        pltpu.sync_copy(out_a, out_hbm.at[0, pl.ds(base, rows_per_tile)])
        pltpu.sync_copy(out_b, out_hbm.at[1, pl.ds(base, rows_per_tile)])

    return sc_call


def kernel(input_ids, attention_mask, W):
    batch = input_ids.shape[0]
    w_flat = jnp.concatenate([W.reshape(4), jnp.zeros((12,), jnp.float32)])
    planes = _make_sc_call(batch)(w_flat)
    return planes.T
